# baseline (speedup 1.0000x reference)
import functools

import jax
import jax.numpy as jnp
import numpy as np
from jax import lax
from jax.experimental import pallas as pl
from jax.experimental.pallas import tpu as pltpu
from jax.experimental.pallas import tpu_sc as plsc

N_NODES = 10000
N_EDGES = 5000
NNZ = 320000
D = 128

NUM_TILES = 32
CHUNK = 128
CHUNKS_PER_TILE = 80
NNZ_PAD = NUM_TILES * CHUNKS_PER_TILE * CHUNK

E_PAD = 5120
N_PAD = 10112

_BN_INV = 1.0 / float(np.sqrt(1.0 + 1e-5))


A_CHUNKS = 160
B_CHUNKS = 0
PIECE = 32


def _sc_segment_sum(table, idx_g, idx_s, s_pad, n_buf):
  mesh = plsc.VectorSubcoreMesh(core_axis_name="c", subcore_axis_name="s")
  zero_blk = jnp.zeros((CHUNK, D), jnp.float32)
  n_zero_chunks = s_pad // CHUNK
  assert PIECE % n_buf == 0

  @functools.partial(
      pl.kernel,
      out_type=jax.ShapeDtypeStruct((2 * s_pad, D), jnp.float32),
      mesh=mesh,
      scratch_types=[
          pltpu.VMEM((PIECE, CHUNK), jnp.int32),
          pltpu.VMEM((PIECE, CHUNK), jnp.int32),
          pltpu.VMEM((n_buf, CHUNK, D), jnp.float32),
          pltpu.VMEM_SHARED((s_pad, D), jnp.float32),
      ] + [pltpu.SemaphoreType.DMA] * n_buf,
  )
  def sc_kernel(table_hbm, idxg_hbm, idxs_hbm, zeros_hbm, out_hbm,
                idxg_v, idxs_v, rows_v, acc_sh, *sems):
    cid = lax.axis_index("c")
    sid = lax.axis_index("s")
    base = jnp.where(cid == 0, sid * A_CHUNKS,
                     16 * A_CHUNKS + sid * B_CHUNKS)
    n_pieces_mine = jnp.where(cid == 0, A_CHUNKS // PIECE, B_CHUNKS // PIECE)

    @pl.loop(0, (n_zero_chunks + 15) // 16)
    def _(j):
      c = j * 16 + sid

      @pl.when(c < n_zero_chunks)
      def _():
        pltpu.sync_copy(zeros_hbm, acc_sh.at[pl.ds(c * CHUNK, CHUNK)])

    plsc.subcore_barrier()

    def gather(j, b):
      pltpu.async_copy(table_hbm.at[idxg_v.at[j]], rows_v.at[b], sems[b])

    def gwait(j, b):
      pltpu.make_async_copy(table_hbm.at[idxg_v.at[j]], rows_v.at[b],
                            sems[b]).wait()

    def scat(j, b):
      pltpu.sync_copy(rows_v.at[b], acc_sh.at[idxs_v.at[j]], add=True)

    @pl.loop(0, A_CHUNKS // PIECE)
    def _(p):
      @pl.when(p < n_pieces_mine)
      def _():
        pltpu.sync_copy(idxg_hbm.at[pl.ds(base + p * PIECE, PIECE)], idxg_v)
        pltpu.sync_copy(idxs_hbm.at[pl.ds(base + p * PIECE, PIECE)], idxs_v)

        for b in range(n_buf):
          gather(b, b)

        @pl.loop(0, PIECE // n_buf - 1)
        def _(h):
          j0 = n_buf * h
          for b in range(n_buf):
            gwait(j0 + b, b)
            scat(j0 + b, b)
            gather(j0 + n_buf + b, b)

        j_last = PIECE - n_buf
        for b in range(n_buf):
          gwait(j_last + b, b)
          scat(j_last + b, b)

    plsc.subcore_barrier()

    @pl.loop(0, (n_zero_chunks + 15) // 16)
    def _(j):
      c = j * 16 + sid

      @pl.when(c < n_zero_chunks)
      def _():
        pltpu.sync_copy(acc_sh.at[pl.ds(c * CHUNK, CHUNK)],
                        out_hbm.at[pl.ds(cid * s_pad + c * CHUNK, CHUNK)])

  return sc_kernel(table, idx_g, idx_s, zero_blk)


def _tc_node_messages(x_0, W_nm, b_nm):
  B = 1000

  def body(x_ref, w_ref, b_ref, o_ref):
    z = jnp.dot(x_ref[...], w_ref[...], preferred_element_type=jnp.float32)
    o_ref[...] = jax.nn.sigmoid(z + b_ref[...])

  return pl.pallas_call(
      body,
      grid=(N_NODES // B,),
      in_specs=[
          pl.BlockSpec((B, D), lambda i: (i, 0)),
          pl.BlockSpec((D, D), lambda i: (0, 0)),
          pl.BlockSpec((1, D), lambda i: (0, 0)),
      ],
      out_specs=pl.BlockSpec((B, D), lambda i: (i, 0)),
      out_shape=jax.ShapeDtypeStruct((N_NODES, D), jnp.float32),
  )(x_0, W_nm, b_nm.reshape(1, D))


def _tc_edge_stage(x_1, parts_e, W_top, W_bot, b_hm, gamma1, beta1):
  B = 1000

  def body(x_ref, p_ref, wt_ref, wb_ref, b_ref, g_ref, be_ref,
           msg_ref, out1_ref):
    agg = p_ref[0] + p_ref[1]
    x = x_ref[...]
    z = jnp.dot(x, wt_ref[...], preferred_element_type=jnp.float32)
    z += jnp.dot(agg, wb_ref[...], preferred_element_type=jnp.float32)
    msg_ref[...] = jax.nn.sigmoid(z + b_ref[...])
    out1_ref[...] = jax.nn.sigmoid(x * (g_ref[...] * _BN_INV) + be_ref[...]
                                   + agg)

  return pl.pallas_call(
      body,
      grid=(N_EDGES // B,),
      in_specs=[
          pl.BlockSpec((B, D), lambda i: (i, 0)),
          pl.BlockSpec((2, B, D), lambda i: (0, i, 0)),
          pl.BlockSpec((D, D), lambda i: (0, 0)),
          pl.BlockSpec((D, D), lambda i: (0, 0)),
          pl.BlockSpec((1, D), lambda i: (0, 0)),
          pl.BlockSpec((1, D), lambda i: (0, 0)),
          pl.BlockSpec((1, D), lambda i: (0, 0)),
      ],
      out_specs=[
          pl.BlockSpec((B, D), lambda i: (i, 0)),
          pl.BlockSpec((B, D), lambda i: (i, 0)),
      ],
      out_shape=[
          jax.ShapeDtypeStruct((N_EDGES, D), jnp.float32),
          jax.ShapeDtypeStruct((N_EDGES, D), jnp.float32),
      ],
  )(x_1, parts_e, W_top, W_bot, b_hm.reshape(1, D), gamma1.reshape(1, D),
    beta1.reshape(1, D))


def _tc_node_update(x_0, parts_n, gamma0, beta0):
  B = 1000

  def body(x_ref, p_ref, g_ref, be_ref, o_ref):
    agg = p_ref[0] + p_ref[1]
    o_ref[...] = jax.nn.sigmoid(x_ref[...] * (g_ref[...] * _BN_INV)
                                + be_ref[...] + agg)

  return pl.pallas_call(
      body,
      grid=(N_NODES // B,),
      in_specs=[
          pl.BlockSpec((B, D), lambda i: (i, 0)),
          pl.BlockSpec((2, B, D), lambda i: (0, i, 0)),
          pl.BlockSpec((1, D), lambda i: (0, 0)),
          pl.BlockSpec((1, D), lambda i: (0, 0)),
      ],
      out_specs=pl.BlockSpec((B, D), lambda i: (i, 0)),
      out_shape=jax.ShapeDtypeStruct((N_NODES, D), jnp.float32),
  )(x_0, parts_n, gamma0.reshape(1, D), beta0.reshape(1, D))


def kernel(x_0, x_1, incidence_indices, W_nm, b_nm, W_hm, b_hm,
           gamma0, beta0, gamma1, beta1):
  src = incidence_indices[0]
  tgt = incidence_indices[1]
  pad = NNZ_PAD - NNZ

  zpad = jnp.zeros((pad,), jnp.int32)
  src_g = jnp.concatenate([src, zpad]).reshape(-1, CHUNK)
  tgt_s = jnp.concatenate([tgt, jnp.full((pad,), N_EDGES, jnp.int32)]
                          ).reshape(-1, CHUNK)
  tgt_g = jnp.concatenate([tgt, zpad]).reshape(-1, CHUNK)
  src_s = jnp.concatenate([src, jnp.full((pad,), N_NODES, jnp.int32)]
                          ).reshape(-1, CHUNK)

  W_top = W_hm[:D]
  W_bot = W_hm[D:]

  x_msg_nodes = _tc_node_messages(x_0, W_nm, b_nm)
  parts_e = _sc_segment_sum(x_msg_nodes, src_g, tgt_s, E_PAD,
                            n_buf=4)
  x_msg_edges, out_x1 = _tc_edge_stage(
      x_1, parts_e.reshape(2, E_PAD, D), W_top, W_bot, b_hm, gamma1, beta1)
  parts_n = _sc_segment_sum(x_msg_edges, tgt_g, src_s, N_PAD,
                            n_buf=2)
  out_x0 = _tc_node_update(x_0, parts_n.reshape(2, N_PAD, D), gamma0, beta0)
  return out_x0, out_x1


# speedup vs baseline: 4.1157x; 4.1157x over previous
import functools

import jax
import jax.numpy as jnp
import numpy as np
from jax import lax
from jax.experimental import pallas as pl
from jax.experimental.pallas import tpu as pltpu
from jax.experimental.pallas import tpu_sc as plsc

N_NODES = 10000
N_EDGES = 5000
NNZ = 320000
D = 128

NUM_TILES = 32
CHUNK = 128
CHUNKS_PER_TILE = 80
NNZ_PAD = NUM_TILES * CHUNKS_PER_TILE * CHUNK

E_PAD = 5120
N_PAD = 10112

_BN_INV = 1.0 / float(np.sqrt(1.0 + 1e-5))


def _sc_segment_sum(table, idx_g, idx_s, s_pad, n_buf, piece):
  mesh = plsc.VectorSubcoreMesh(core_axis_name="c", subcore_axis_name="s")
  zero_blk = jnp.zeros((CHUNK, D), jnp.float32)
  n_zero_chunks = s_pad // CHUNK
  n_pieces = CHUNKS_PER_TILE // piece
  assert CHUNKS_PER_TILE % piece == 0 and piece % n_buf == 0

  @functools.partial(
      pl.kernel,
      out_type=jax.ShapeDtypeStruct((2 * s_pad, D), jnp.float32),
      mesh=mesh,
      scratch_types=[
          pltpu.VMEM((piece, CHUNK), jnp.int32),
          pltpu.VMEM((piece, CHUNK), jnp.int32),
          pltpu.VMEM((n_buf, CHUNK, D), jnp.float32),
          pltpu.VMEM_SHARED((s_pad, D), jnp.float32),
      ] + [pltpu.SemaphoreType.DMA] * n_buf,
  )
  def sc_kernel(table_hbm, idxg_hbm, idxs_hbm, zeros_hbm, out_hbm,
                idxg_v, idxs_v, rows_v, acc_sh, *sems):
    cid = lax.axis_index("c")
    sid = lax.axis_index("s")
    wid = sid * 2 + cid

    @pl.loop(0, (n_zero_chunks + 15) // 16)
    def _(j):
      c = j * 16 + sid

      @pl.when(c < n_zero_chunks)
      def _():
        pltpu.sync_copy(zeros_hbm, acc_sh.at[pl.ds(c * CHUNK, CHUNK)])

    def gather(j, b):
      pltpu.async_copy(table_hbm.at[idxg_v.at[j]], rows_v.at[b], sems[b])

    def gwait(j, b):
      pltpu.make_async_copy(table_hbm.at[idxg_v.at[j]], rows_v.at[b],
                            sems[b]).wait()

    def scat(j, b):
      pltpu.sync_copy(rows_v.at[b], acc_sh.at[idxs_v.at[j]], add=True)

    barriered = False
    for p in range(n_pieces):
      base = wid * CHUNKS_PER_TILE + p * piece
      pltpu.sync_copy(idxg_hbm.at[pl.ds(base, piece)], idxg_v)
      pltpu.sync_copy(idxs_hbm.at[pl.ds(base, piece)], idxs_v)

      if not barriered:
        plsc.subcore_barrier()
        barriered = True

      for b in range(n_buf):
        gather(b, b)

      @pl.loop(0, piece // n_buf - 1)
      def _(h):
        j0 = n_buf * h
        for b in range(n_buf):
          gwait(j0 + b, b)
          scat(j0 + b, b)
          gather(j0 + n_buf + b, b)

      j_last = piece - n_buf
      for b in range(n_buf):
        gwait(j_last + b, b)
        scat(j_last + b, b)

    plsc.subcore_barrier()

    @pl.loop(0, (n_zero_chunks + 15) // 16)
    def _(j):
      c = j * 16 + sid

      @pl.when(c < n_zero_chunks)
      def _():
        pltpu.sync_copy(acc_sh.at[pl.ds(c * CHUNK, CHUNK)],
                        out_hbm.at[pl.ds(cid * s_pad + c * CHUNK, CHUNK)])

  return sc_kernel(table, idx_g, idx_s, zero_blk)


def _tc_node_messages(x_0, W_nm, b_nm):
  B = 1000

  def body(x_ref, w_ref, b_ref, o_ref):
    z = jnp.dot(x_ref[...], w_ref[...], preferred_element_type=jnp.float32)
    o_ref[...] = jax.nn.sigmoid(z + b_ref[...])

  return pl.pallas_call(
      body,
      grid=(N_NODES // B,),
      in_specs=[
          pl.BlockSpec((B, D), lambda i: (i, 0)),
          pl.BlockSpec((D, D), lambda i: (0, 0)),
          pl.BlockSpec((1, D), lambda i: (0, 0)),
      ],
      out_specs=pl.BlockSpec((B, D), lambda i: (i, 0)),
      out_shape=jax.ShapeDtypeStruct((N_NODES, D), jnp.float32),
  )(x_0, W_nm, b_nm.reshape(1, D))


def _tc_edge_stage(x_1, parts_e, W_top, W_bot, b_hm, gamma1, beta1):
  B = 1000

  def body(x_ref, p_ref, wt_ref, wb_ref, b_ref, g_ref, be_ref,
           msg_ref, out1_ref):
    agg = p_ref[0] + p_ref[1]
    x = x_ref[...]
    z = jnp.dot(x, wt_ref[...], preferred_element_type=jnp.float32)
    z += jnp.dot(agg, wb_ref[...], preferred_element_type=jnp.float32)
    msg_ref[...] = jax.nn.sigmoid(z + b_ref[...])
    out1_ref[...] = jax.nn.sigmoid(x * (g_ref[...] * _BN_INV) + be_ref[...]
                                   + agg)

  return pl.pallas_call(
      body,
      grid=(N_EDGES // B,),
      in_specs=[
          pl.BlockSpec((B, D), lambda i: (i, 0)),
          pl.BlockSpec((2, B, D), lambda i: (0, i, 0)),
          pl.BlockSpec((D, D), lambda i: (0, 0)),
          pl.BlockSpec((D, D), lambda i: (0, 0)),
          pl.BlockSpec((1, D), lambda i: (0, 0)),
          pl.BlockSpec((1, D), lambda i: (0, 0)),
          pl.BlockSpec((1, D), lambda i: (0, 0)),
      ],
      out_specs=[
          pl.BlockSpec((B, D), lambda i: (i, 0)),
          pl.BlockSpec((B, D), lambda i: (i, 0)),
      ],
      out_shape=[
          jax.ShapeDtypeStruct((N_EDGES, D), jnp.float32),
          jax.ShapeDtypeStruct((N_EDGES, D), jnp.float32),
      ],
  )(x_1, parts_e, W_top, W_bot, b_hm.reshape(1, D), gamma1.reshape(1, D),
    beta1.reshape(1, D))


def _tc_node_update(x_0, parts_n, gamma0, beta0):
  B = 1000

  def body(x_ref, p_ref, g_ref, be_ref, o_ref):
    agg = p_ref[0] + p_ref[1]
    o_ref[...] = jax.nn.sigmoid(x_ref[...] * (g_ref[...] * _BN_INV)
                                + be_ref[...] + agg)

  return pl.pallas_call(
      body,
      grid=(N_NODES // B,),
      in_specs=[
          pl.BlockSpec((B, D), lambda i: (i, 0)),
          pl.BlockSpec((2, B, D), lambda i: (0, i, 0)),
          pl.BlockSpec((1, D), lambda i: (0, 0)),
          pl.BlockSpec((1, D), lambda i: (0, 0)),
      ],
      out_specs=pl.BlockSpec((B, D), lambda i: (i, 0)),
      out_shape=jax.ShapeDtypeStruct((N_NODES, D), jnp.float32),
  )(x_0, parts_n, gamma0.reshape(1, D), beta0.reshape(1, D))


def kernel(x_0, x_1, incidence_indices, W_nm, b_nm, W_hm, b_hm,
           gamma0, beta0, gamma1, beta1):
  src = incidence_indices[0]
  tgt = incidence_indices[1]
  pad = NNZ_PAD - NNZ

  ar = jnp.arange(pad, dtype=jnp.int32)
  gpad_n = ar % N_NODES
  gpad_e = ar % N_EDGES
  spad_e = N_EDGES + ar % (E_PAD - N_EDGES)
  spad_n = N_NODES + ar % (N_PAD - N_NODES)
  src_g = jnp.concatenate([src, gpad_n]).reshape(-1, CHUNK)
  tgt_s = jnp.concatenate([tgt, spad_e]).reshape(-1, CHUNK)
  tgt_g = jnp.concatenate([tgt, gpad_e]).reshape(-1, CHUNK)
  src_s = jnp.concatenate([src, spad_n]).reshape(-1, CHUNK)

  W_top = W_hm[:D]
  W_bot = W_hm[D:]

  x_msg_nodes = _tc_node_messages(x_0, W_nm, b_nm)
  parts_e = _sc_segment_sum(x_msg_nodes, src_g, tgt_s, E_PAD,
                            n_buf=4, piece=80)
  x_msg_edges, out_x1 = _tc_edge_stage(
      x_1, parts_e.reshape(2, E_PAD, D), W_top, W_bot, b_hm, gamma1, beta1)
  parts_n = _sc_segment_sum(x_msg_edges, tgt_g, src_s, N_PAD,
                            n_buf=2, piece=40)
  out_x0 = _tc_node_update(x_0, parts_n.reshape(2, N_PAD, D), gamma0, beta0)
  return out_x0, out_x1
